# baseline (speedup 1.0000x reference)
import functools

import jax
import jax.numpy as jnp
from jax import lax
from jax.experimental import pallas as pl
from jax.experimental.pallas import tpu as pltpu
from jax.experimental.pallas import tpu_sc as plsc

_B = 16384
_L = 50
_F = 32
_N = _B * _L
_NW = 32
_PER_W = _N // _NW
_CHUNK = 1024
_NCHUNK = _PER_W // _CHUNK
_G = 128
_NG = _CHUNK // _G


def _make_sc_kernel():
    mesh = plsc.VectorSubcoreMesh(core_axis_name="c", subcore_axis_name="s")

    @functools.partial(
        pl.kernel,
        mesh=mesh,
        out_type=[
            jax.ShapeDtypeStruct((_N, _F), jnp.float32),
            jax.ShapeDtypeStruct((_N, _F), jnp.int32),
        ],
        scratch_types=[
            pltpu.VMEM((_NG, _G), jnp.int32),
            pltpu.VMEM((_CHUNK, _F), jnp.float32),
            pltpu.VMEM((_CHUNK, _F), jnp.int32),
            pltpu.SemaphoreType.DMA,
        ],
        compiler_params=pltpu.CompilerParams(use_tc_tiling_on_sc=False),
    )
    def body(x_hbm, table_hbm, out_hbm, mask_hbm,
             idx_v, rows_v, mask_v, sem):
        wid = lax.axis_index("s") * 2 + lax.axis_index("c")

        def chunk(c, carry):
            rb = wid * (_PER_W // _G) + c * _NG
            fb = wid * _PER_W + c * _CHUNK
            pltpu.sync_copy(x_hbm.at[pl.ds(rb, _NG)], idx_v)
            cps = [
                pltpu.async_copy(
                    table_hbm.at[idx_v.at[j]],
                    rows_v.at[pl.ds(j * _G, _G)],
                    sem,
                )
                for j in range(_NG)
            ]
            for cp in cps:
                cp.wait()

            def group(g, cc):
                iv = idx_v[g // 8, pl.ds(lax.rem(g, 8) * 16, 16)]
                mi_g = jnp.where(iv != 0, 1, 0).astype(jnp.int32)
                for k in range(16):
                    r = g * 16 + k
                    mi = jnp.full((16,), mi_g[k], jnp.int32)
                    mf = mi.astype(jnp.float32)
                    rows_v[r, pl.ds(0, 16)] = rows_v[r, pl.ds(0, 16)] * mf
                    rows_v[r, pl.ds(16, 16)] = rows_v[r, pl.ds(16, 16)] * mf
                    mask_v[r, pl.ds(0, 16)] = mi
                    mask_v[r, pl.ds(16, 16)] = mi
                return cc

            lax.fori_loop(0, _CHUNK // 16, group, 0)
            pltpu.sync_copy(rows_v, out_hbm.at[pl.ds(fb, _CHUNK)])
            pltpu.sync_copy(mask_v, mask_hbm.at[pl.ds(fb, _CHUNK)])
            return carry

        lax.fori_loop(0, _NCHUNK, chunk, 0)

    return body


_sc_kernel = _make_sc_kernel()


def kernel(x, table):
    x2d = x.reshape(_N // _G, _G)
    out_flat, mask_flat = _sc_kernel(x2d, table)
    return (out_flat.reshape(_B, _L, _F), mask_flat.reshape(_B, _L, _F))


# speedup vs baseline: 1.0674x; 1.0674x over previous
import functools

import jax
import jax.numpy as jnp
from jax import lax
from jax.experimental import pallas as pl
from jax.experimental.pallas import tpu as pltpu
from jax.experimental.pallas import tpu_sc as plsc

_B = 16384
_L = 50
_F = 32
_N = _B * _L
_NW = 32
_PER_W = _N // _NW
_CHUNK = 1024
_NCHUNK = _PER_W // _CHUNK
_G = 128
_NG = _CHUNK // _G
_NGRP = _CHUNK // 16


def _make_sc_kernel():
    mesh = plsc.VectorSubcoreMesh(core_axis_name="c", subcore_axis_name="s")

    @functools.partial(
        pl.kernel,
        mesh=mesh,
        out_type=[
            jax.ShapeDtypeStruct((_N, _F), jnp.float32),
            jax.ShapeDtypeStruct((_N, _F), jnp.int32),
        ],
        scratch_types=[
            pltpu.VMEM((2, _NG, _G), jnp.int32),
            pltpu.VMEM((2, _CHUNK, _F), jnp.float32),
            pltpu.VMEM((_CHUNK, _F), jnp.int32),
            pltpu.SemaphoreType.DMA,
        ],
        compiler_params=pltpu.CompilerParams(use_tc_tiling_on_sc=False),
    )
    def body(x_hbm, table_hbm, out_hbm, mask_hbm,
             idx_v, rows_v, mask_v, sem):
        wid = lax.axis_index("s") * 2 + lax.axis_index("c")

        ones_i = jnp.ones((16,), jnp.int32)

        def init(r, cc):
            mask_v[r, pl.ds(0, 16)] = ones_i
            mask_v[r, pl.ds(16, 16)] = ones_i
            return cc

        lax.fori_loop(0, _CHUNK, init, 0)

        def finish_chunk(pbuf, cprev, zero_prev):
            fbp = wid * _PER_W + cprev * _CHUNK

            @pl.when(zero_prev != 0)
            def _():
                def fix(g, cc):
                    iv = idx_v[pbuf, g // 8, pl.ds(lax.rem(g, 8) * 16, 16)]
                    mi_g = jnp.where(iv != 0, 1, 0).astype(jnp.int32)
                    for k in range(16):
                        r = g * 16 + k
                        mi = jnp.full((16,), mi_g[k], jnp.int32)
                        mf = mi.astype(jnp.float32)
                        rows_v[pbuf, r, pl.ds(0, 16)] = (
                            rows_v[pbuf, r, pl.ds(0, 16)] * mf)
                        rows_v[pbuf, r, pl.ds(16, 16)] = (
                            rows_v[pbuf, r, pl.ds(16, 16)] * mf)
                        mask_v[r, pl.ds(0, 16)] = mi
                        mask_v[r, pl.ds(16, 16)] = mi
                    return cc

                lax.fori_loop(0, _NGRP, fix, 0)

            pltpu.sync_copy(rows_v.at[pbuf], out_hbm.at[pl.ds(fbp, _CHUNK)])
            pltpu.sync_copy(mask_v, mask_hbm.at[pl.ds(fbp, _CHUNK)])

            @pl.when(zero_prev != 0)
            def _():
                lax.fori_loop(0, _CHUNK, init, 0)

        def step(c, zero_prev):
            buf = lax.rem(c, 2)
            pbuf = 1 - buf

            rb = wid * (_PER_W // _G) + c * _NG
            pltpu.sync_copy(x_hbm.at[pl.ds(rb, _NG)], idx_v.at[buf])
            cps = [
                pltpu.async_copy(
                    table_hbm.at[idx_v.at[buf].at[j]],
                    rows_v.at[buf].at[pl.ds(j * _G, _G)],
                    sem,
                )
                for j in range(_NG)
            ]

            @pl.when(c >= 1)
            def _():
                finish_chunk(pbuf, c - 1, zero_prev)

            for cp in cps:
                cp.wait()
            mv = idx_v[buf, 0, pl.ds(0, 16)]
            for g in range(1, _NGRP):
                mv = jnp.minimum(
                    mv, idx_v[buf, g // 8, pl.ds(lax.rem(g, 8) * 16, 16)])
            zmin = mv[0]
            for k in range(1, 16):
                zmin = jnp.minimum(zmin, mv[k])
            return jnp.where(zmin == 0, 1, 0).astype(jnp.int32)

        zero_last = lax.fori_loop(0, _NCHUNK, step, jnp.int32(0))
        finish_chunk((_NCHUNK - 1) % 2, _NCHUNK - 1, zero_last)

    return body


_sc_kernel = _make_sc_kernel()


def kernel(x, table):
    x2d = x.reshape(_N // _G, _G)
    out_flat, mask_flat = _sc_kernel(x2d, table)
    return (out_flat.reshape(_B, _L, _F), mask_flat.reshape(_B, _L, _F))


# speedup vs baseline: 1.8962x; 1.7764x over previous
import functools

import jax
import jax.numpy as jnp
from jax import lax
from jax.experimental import pallas as pl
from jax.experimental.pallas import tpu as pltpu
from jax.experimental.pallas import tpu_sc as plsc

_B = 16384
_L = 50
_F = 32
_NW = 32
_BPW = _B // _NW
_CB = 16
_NCHUNK = _BPW // _CB
_OFFS = (0, 16, 32, 34)


def _make_sc_kernel():
    mesh = plsc.VectorSubcoreMesh(core_axis_name="c", subcore_axis_name="s")

    @functools.partial(
        pl.kernel,
        mesh=mesh,
        out_type=[
            jax.ShapeDtypeStruct((_B, _L, _F), jnp.float32),
            jax.ShapeDtypeStruct((_B, _L, _F), jnp.int32),
        ],
        scratch_types=[
            pltpu.VMEM((2, _CB, _L), jnp.int32),
            pltpu.VMEM((2, _CB, _L, _F), jnp.float32),
            pltpu.VMEM((_CB, _L, _F), jnp.int32),
            pltpu.SemaphoreType.DMA,
        ],
        compiler_params=pltpu.CompilerParams(use_tc_tiling_on_sc=False),
    )
    def body(x_hbm, table_hbm, out_hbm, mask_hbm,
             idx_v, rows_v, mask_v, sem):
        wid = lax.axis_index("s") * 2 + lax.axis_index("c")

        ones_i = jnp.ones((16,), jnp.int32)

        def init(r, cc):
            mask_v[r // _L, lax.rem(r, _L), pl.ds(0, 16)] = ones_i
            mask_v[r // _L, lax.rem(r, _L), pl.ds(16, 16)] = ones_i
            return cc

        lax.fori_loop(0, _CB * _L, init, 0)

        def finish_chunk(pbuf, cprev, zero_prev):
            b0 = wid * _BPW + cprev * _CB

            @pl.when(zero_prev != 0)
            def _():
                def fix(j, cc):
                    for o in _OFFS:
                        iv = idx_v[pbuf, j, pl.ds(o, 16)]
                        mi_g = jnp.where(iv != 0, 1, 0).astype(jnp.int32)
                        for k in range(16):
                            ll = o + k
                            mi = jnp.full((16,), mi_g[k], jnp.int32)
                            mf = mi.astype(jnp.float32)
                            rows_v[pbuf, j, ll, pl.ds(0, 16)] = (
                                rows_v[pbuf, j, ll, pl.ds(0, 16)] * mf)
                            rows_v[pbuf, j, ll, pl.ds(16, 16)] = (
                                rows_v[pbuf, j, ll, pl.ds(16, 16)] * mf)
                            mask_v[j, ll, pl.ds(0, 16)] = mi
                            mask_v[j, ll, pl.ds(16, 16)] = mi
                    return cc

                lax.fori_loop(0, _CB, fix, 0)

            pltpu.sync_copy(rows_v.at[pbuf], out_hbm.at[pl.ds(b0, _CB)])
            pltpu.sync_copy(mask_v, mask_hbm.at[pl.ds(b0, _CB)])

            @pl.when(zero_prev != 0)
            def _():
                lax.fori_loop(0, _CB * _L, init, 0)

        def step(c, zero_prev):
            buf = lax.rem(c, 2)
            pbuf = 1 - buf

            b0 = wid * _BPW + c * _CB
            pltpu.sync_copy(x_hbm.at[pl.ds(b0, _CB)], idx_v.at[buf])
            cps = [
                pltpu.async_copy(
                    table_hbm.at[idx_v.at[buf].at[j]],
                    rows_v.at[buf].at[j],
                    sem,
                )
                for j in range(_CB)
            ]

            @pl.when(c >= 1)
            def _():
                finish_chunk(pbuf, c - 1, zero_prev)

            for cp in cps:
                cp.wait()
            mv = idx_v[buf, 0, pl.ds(0, 16)]
            first = True
            for j in range(_CB):
                for o in _OFFS:
                    if first:
                        first = False
                        continue
                    mv = jnp.minimum(mv, idx_v[buf, j, pl.ds(o, 16)])
            zmin = mv[0]
            for k in range(1, 16):
                zmin = jnp.minimum(zmin, mv[k])
            return jnp.where(zmin == 0, 1, 0).astype(jnp.int32)

        zero_last = lax.fori_loop(0, _NCHUNK, step, jnp.int32(0))
        finish_chunk((_NCHUNK - 1) % 2, _NCHUNK - 1, zero_last)

    return body


_sc_kernel = _make_sc_kernel()


def kernel(x, table):
    out, mask = _sc_kernel(x, table)
    return (out, mask)


# speedup vs baseline: 2.1616x; 1.1400x over previous
import functools

import jax
import jax.numpy as jnp
from jax import lax
from jax.experimental import pallas as pl
from jax.experimental.pallas import tpu as pltpu
from jax.experimental.pallas import tpu_sc as plsc

_B = 16384
_L = 50
_F = 32
_NW = 32
_NBB = _B // 128
_BBW = _NBB // _NW
_NSTEP = _BBW * _L


def _make_sc_kernel():
    mesh = plsc.VectorSubcoreMesh(core_axis_name="c", subcore_axis_name="s")

    @functools.partial(
        pl.kernel,
        mesh=mesh,
        out_type=[
            jax.ShapeDtypeStruct((_L, _F // 8, _NBB, 8, 128), jnp.float32),
            jax.ShapeDtypeStruct((_L, _F // 8, _NBB, 8, 128), jnp.int32),
        ],
        scratch_types=[
            pltpu.VMEM((_L, 128), jnp.int32),
            pltpu.VMEM((2, 128, _F), jnp.float32),
            pltpu.VMEM((_F // 8, 8, 128), jnp.float32),
            pltpu.VMEM((_F // 8, 8, 128), jnp.int32),
            pltpu.VMEM((_F // 8, 8, 128), jnp.int32),
            pltpu.SemaphoreType.DMA,
            pltpu.SemaphoreType.DMA,
        ],
        compiler_params=pltpu.CompilerParams(
            use_tc_tiling_on_sc=False, needs_layout_passes=False),
    )
    def body(xt_hbm, table_hbm, out_hbm, mask_hbm,
             idx_v, rows_v, tile_v, ones_v, maskt_v, sem0, sem1):
        wid = lax.axis_index("s") * 2 + lax.axis_index("c")
        sems = (sem0, sem1)

        ones_i = jnp.ones((16,), jnp.int32)

        def init(r, cc):
            for h in range(8):
                ones_v[r // 8, lax.rem(r, 8), pl.ds(h * 16, 16)] = ones_i
            return cc

        lax.fori_loop(0, _F, init, 0)

        def load_slab(bb):
            pltpu.sync_copy(
                xt_hbm.at[:, pl.ds((wid * _BBW + bb) * 128, 128)], idx_v)

        def fire(s, buf):
            pltpu.async_copy(table_hbm.at[idx_v.at[lax.rem(s, _L)]],
                             rows_v.at[buf], sems[buf])

        def land(s, buf):
            pltpu.make_async_copy(table_hbm.at[idx_v.at[lax.rem(s, _L)]],
                                  rows_v.at[buf], sems[buf]).wait()

        def detect(l):
            mv = idx_v[l, pl.ds(0, 16)]
            for h in range(1, 8):
                mv = jnp.minimum(mv, idx_v[l, pl.ds(h * 16, 16)])
            zmin = mv[0]
            for k in range(1, 16):
                zmin = jnp.minimum(zmin, mv[k])
            return jnp.where(zmin == 0, 1, 0).astype(jnp.int32)

        def emit(s, buf):
            bb = s // _L
            l = lax.rem(s, _L)
            gb = wid * _BBW + bb
            z = detect(l)

            @pl.when(z == 0)
            def _():
                for f in range(_F):
                    for h in range(8):
                        rr = h * 16 + jnp.arange(16, dtype=jnp.int32)
                        cc = jnp.full((16,), f, jnp.int32)
                        tile_v[f // 8, f % 8, pl.ds(h * 16, 16)] = (
                            plsc.load_gather(rows_v.at[buf], [rr, cc]))

            @pl.when(z != 0)
            def _():
                mfs = []
                for h in range(8):
                    iv = idx_v[l, pl.ds(h * 16, 16)]
                    mi = jnp.where(iv != 0, 1, 0).astype(jnp.int32)
                    mfs.append((mi, mi.astype(jnp.float32)))
                for f in range(_F):
                    for h in range(8):
                        rr = h * 16 + jnp.arange(16, dtype=jnp.int32)
                        cc = jnp.full((16,), f, jnp.int32)
                        tile_v[f // 8, f % 8, pl.ds(h * 16, 16)] = (
                            plsc.load_gather(rows_v.at[buf], [rr, cc])
                            * mfs[h][1])
                        maskt_v[f // 8, f % 8, pl.ds(h * 16, 16)] = mfs[h][0]

            pltpu.sync_copy(tile_v, out_hbm.at[l, :, gb])

            @pl.when(z == 0)
            def _():
                pltpu.sync_copy(ones_v, mask_hbm.at[l, :, gb])

            @pl.when(z != 0)
            def _():
                pltpu.sync_copy(maskt_v, mask_hbm.at[l, :, gb])

        load_slab(0)
        fire(0, 0)

        def pair(t, carry):
            s0 = 2 * t
            s1 = s0 + 1

            @pl.when(lax.rem(s1, _L) != 0)
            def _():
                fire(s1, 1)

            land(s0, 0)
            emit(s0, 0)

            @pl.when(lax.rem(s1, _L) == 0)
            def _():
                load_slab(s1 // _L)
                fire(s1, 1)

            @pl.when((s1 + 1 < _NSTEP) & (lax.rem(s1 + 1, _L) != 0))
            def _():
                fire(s1 + 1, 0)

            land(s1, 1)
            emit(s1, 1)

            @pl.when((s1 + 1 < _NSTEP) & (lax.rem(s1 + 1, _L) == 0))
            def _():
                load_slab((s1 + 1) // _L)
                fire(s1 + 1, 0)

            return carry

        lax.fori_loop(0, _NSTEP // 2, pair, jnp.int32(0))

    return body


_sc_kernel = _make_sc_kernel()


def kernel(x, table):
    xt = jnp.swapaxes(x, 0, 1)
    out5, mask5 = _sc_kernel(xt, table)
    out = out5.transpose(2, 4, 0, 1, 3).reshape(_B, _L, _F)
    mask = mask5.transpose(2, 4, 0, 1, 3).reshape(_B, _L, _F)
    return (out, mask)


# speedup vs baseline: 3.2010x; 1.4808x over previous
import functools

import jax
import jax.numpy as jnp
from jax import lax
from jax.experimental import pallas as pl
from jax.experimental.pallas import tpu as pltpu
from jax.experimental.pallas import tpu_sc as plsc

_B = 16384
_L = 50
_F = 32
_NW = 32
_NBB = _B // 128
_BBW = _NBB // _NW
_NSTEP = _BBW * _L


def _make_sc_kernel():
    mesh = plsc.VectorSubcoreMesh(core_axis_name="c", subcore_axis_name="s")

    @functools.partial(
        pl.kernel,
        mesh=mesh,
        out_type=[
            jax.ShapeDtypeStruct((_L, _F // 8, _NBB, 8, 128), jnp.float32),
            jax.ShapeDtypeStruct((_L, _F // 8, _NBB, 8, 128), jnp.int32),
        ],
        scratch_types=[
            pltpu.VMEM((_L, 128), jnp.int32),
            pltpu.VMEM((2, 128, _F), jnp.float32),
            pltpu.VMEM((_F // 8, 8, 129), jnp.float32),
            pltpu.VMEM((_F // 8, 8, 128), jnp.int32),
            pltpu.VMEM((_F // 8, 8, 129), jnp.int32),
            pltpu.SemaphoreType.DMA,
            pltpu.SemaphoreType.DMA,
        ],
        compiler_params=pltpu.CompilerParams(
            use_tc_tiling_on_sc=False, needs_layout_passes=False),
    )
    def body(xt_hbm, table_hbm, out_hbm, mask_hbm,
             idx_v, rows_v, tile_v, ones_v, maskt_v, sem0, sem1):
        wid = lax.axis_index("s") * 2 + lax.axis_index("c")
        sems = (sem0, sem1)

        ones_i = jnp.ones((16,), jnp.int32)

        def init(r, cc):
            for h in range(8):
                ones_v[r // 8, lax.rem(r, 8), pl.ds(h * 16, 16)] = ones_i
            return cc

        lax.fori_loop(0, _F, init, 0)

        def load_slab(bb):
            pltpu.sync_copy(
                xt_hbm.at[:, pl.ds((wid * _BBW + bb) * 128, 128)], idx_v)

        def fire(s, buf):
            pltpu.async_copy(table_hbm.at[idx_v.at[lax.rem(s, _L)]],
                             rows_v.at[buf], sems[buf])

        def land(s, buf):
            pltpu.make_async_copy(table_hbm.at[idx_v.at[lax.rem(s, _L)]],
                                  rows_v.at[buf], sems[buf]).wait()

        def detect(l):
            mv = idx_v[l, pl.ds(0, 16)]
            for h in range(1, 8):
                mv = jnp.minimum(mv, idx_v[l, pl.ds(h * 16, 16)])
            zmin = mv[0]
            for k in range(1, 16):
                zmin = jnp.minimum(zmin, mv[k])
            return jnp.where(zmin == 0, 1, 0).astype(jnp.int32)

        def emit(s, buf):
            bb = s // _L
            l = lax.rem(s, _L)
            gb = wid * _BBW + bb
            z = detect(l)

            _i16 = jnp.arange(16, dtype=jnp.int32)
            _rg = [( (f0 + _i16) // 8, (f0 + _i16) % 8) for f0 in (0, 16)]

            @pl.when(z == 0)
            def _():
                for b in range(128):
                    cb_ = jnp.full((16,), b, jnp.int32)
                    for hf, (rg, rs) in enumerate(_rg):
                        v = rows_v[buf, b, pl.ds(hf * 16, 16)]
                        plsc.store_scatter(tile_v, [rg, rs, cb_], v)

            @pl.when(z != 0)
            def _():
                mis = []
                for h in range(8):
                    iv = idx_v[l, pl.ds(h * 16, 16)]
                    mis.append(jnp.where(iv != 0, 1, 0).astype(jnp.int32))
                for b in range(128):
                    cb_ = jnp.full((16,), b, jnp.int32)
                    mi_b = jnp.full((16,), mis[b // 16][b % 16], jnp.int32)
                    mf_b = mi_b.astype(jnp.float32)
                    for hf, (rg, rs) in enumerate(_rg):
                        v = rows_v[buf, b, pl.ds(hf * 16, 16)] * mf_b
                        plsc.store_scatter(tile_v, [rg, rs, cb_], v)
                        plsc.store_scatter(maskt_v, [rg, rs, cb_], mi_b)

            pltpu.sync_copy(tile_v.at[:, :, pl.ds(0, 128)],
                            out_hbm.at[l, :, gb])

            @pl.when(z == 0)
            def _():
                pltpu.sync_copy(ones_v, mask_hbm.at[l, :, gb])

            @pl.when(z != 0)
            def _():
                pltpu.sync_copy(maskt_v.at[:, :, pl.ds(0, 128)],
                                mask_hbm.at[l, :, gb])

        load_slab(0)
        fire(0, 0)

        def pair(t, carry):
            s0 = 2 * t
            s1 = s0 + 1

            @pl.when(lax.rem(s1, _L) != 0)
            def _():
                fire(s1, 1)

            land(s0, 0)
            emit(s0, 0)

            @pl.when(lax.rem(s1, _L) == 0)
            def _():
                load_slab(s1 // _L)
                fire(s1, 1)

            @pl.when((s1 + 1 < _NSTEP) & (lax.rem(s1 + 1, _L) != 0))
            def _():
                fire(s1 + 1, 0)

            land(s1, 1)
            emit(s1, 1)

            @pl.when((s1 + 1 < _NSTEP) & (lax.rem(s1 + 1, _L) == 0))
            def _():
                load_slab((s1 + 1) // _L)
                fire(s1 + 1, 0)

            return carry

        lax.fori_loop(0, _NSTEP // 2, pair, jnp.int32(0))

    return body


_sc_kernel = _make_sc_kernel()


def kernel(x, table):
    xt = jnp.swapaxes(x, 0, 1)
    out5, mask5 = _sc_kernel(xt, table)
    out = out5.transpose(2, 4, 0, 1, 3).reshape(_B, _L, _F)
    mask = mask5.transpose(2, 4, 0, 1, 3).reshape(_B, _L, _F)
    return (out, mask)


# speedup vs baseline: 3.5285x; 1.1023x over previous
import functools

import jax
import jax.numpy as jnp
from jax import lax
from jax.experimental import pallas as pl
from jax.experimental.pallas import tpu as pltpu
from jax.experimental.pallas import tpu_sc as plsc

_B = 16384
_L = 50
_F = 32
_NW = 32
_NBB = _B // 128
_BBW = _NBB // _NW
_NSTEP = _BBW * _L


def _make_sc_kernel():
    mesh = plsc.VectorSubcoreMesh(core_axis_name="c", subcore_axis_name="s")

    @functools.partial(
        pl.kernel,
        mesh=mesh,
        out_type=[
            jax.ShapeDtypeStruct((_L, _F // 8, _NBB, 8, 128), jnp.float32),
            jax.ShapeDtypeStruct((_L, _F // 8, _NBB, 8, 128), jnp.int32),
        ],
        scratch_types=[
            pltpu.VMEM((_L, 128), jnp.int32),
            pltpu.VMEM((2, 128, _F), jnp.float32),
            pltpu.VMEM((_F // 8, 8, 129), jnp.float32),
            pltpu.VMEM((_F // 8, 8, 128), jnp.int32),
            pltpu.VMEM((_F // 8, 8, 129), jnp.int32),
            pltpu.SemaphoreType.DMA,
            pltpu.SemaphoreType.DMA,
        ],
        compiler_params=pltpu.CompilerParams(
            use_tc_tiling_on_sc=False, needs_layout_passes=False),
    )
    def body(xt_hbm, table_hbm, out_hbm, mask_hbm,
             idx_v, rows_v, tile_v, ones_v, maskt_v, sem0, sem1):
        wid = lax.axis_index("s") * 2 + lax.axis_index("c")
        sems = (sem0, sem1)

        ones_i = jnp.ones((16,), jnp.int32)

        def init(r, cc):
            for h in range(8):
                ones_v[r // 8, lax.rem(r, 8), pl.ds(h * 16, 16)] = ones_i
            return cc

        lax.fori_loop(0, _F, init, 0)

        def load_slab(bb):
            pltpu.sync_copy(
                xt_hbm.at[:, pl.ds((wid * _BBW + bb) * 128, 128)], idx_v)

        def fire(s, buf):
            pltpu.async_copy(table_hbm.at[idx_v.at[lax.rem(s, _L)]],
                             rows_v.at[buf], sems[buf])

        def land(s, buf):
            pltpu.make_async_copy(table_hbm.at[idx_v.at[lax.rem(s, _L)]],
                                  rows_v.at[buf], sems[buf]).wait()

        def detect(l):
            mv = idx_v[l, pl.ds(0, 16)]
            for h in range(1, 8):
                mv = jnp.minimum(mv, idx_v[l, pl.ds(h * 16, 16)])
            zmin = mv[0]
            for k in range(1, 16):
                zmin = jnp.minimum(zmin, mv[k])
            return jnp.where(zmin == 0, 1, 0).astype(jnp.int32)

        def emit(s, buf):
            bb = s // _L
            l = lax.rem(s, _L)
            gb = wid * _BBW + bb
            z = detect(l)

            _i16 = jnp.arange(16, dtype=jnp.int32)
            _rg = [( (f0 + _i16) // 8, (f0 + _i16) % 8) for f0 in (0, 16)]

            @pl.when(z == 0)
            def _():
                for b0 in range(0, 128, 8):
                    vs = [rows_v[buf, b0 + i, pl.ds(hf * 16, 16)]
                          for i in range(8) for hf in range(2)]
                    for i in range(8):
                        cb_ = jnp.full((16,), b0 + i, jnp.int32)
                        for hf, (rg, rs) in enumerate(_rg):
                            plsc.store_scatter(tile_v, [rg, rs, cb_],
                                               vs[2 * i + hf])

            @pl.when(z != 0)
            def _():
                mis = []
                for h in range(8):
                    iv = idx_v[l, pl.ds(h * 16, 16)]
                    mis.append(jnp.where(iv != 0, 1, 0).astype(jnp.int32))
                for b0 in range(0, 128, 8):
                    vs = [rows_v[buf, b0 + i, pl.ds(hf * 16, 16)]
                          for i in range(8) for hf in range(2)]
                    for i in range(8):
                        b = b0 + i
                        cb_ = jnp.full((16,), b, jnp.int32)
                        mi_b = jnp.full((16,), mis[b // 16][b % 16], jnp.int32)
                        mf_b = mi_b.astype(jnp.float32)
                        for hf, (rg, rs) in enumerate(_rg):
                            plsc.store_scatter(tile_v, [rg, rs, cb_],
                                               vs[2 * i + hf] * mf_b)
                            plsc.store_scatter(maskt_v, [rg, rs, cb_], mi_b)

            pltpu.sync_copy(tile_v.at[:, :, pl.ds(0, 128)],
                            out_hbm.at[l, :, gb])

            @pl.when(z == 0)
            def _():
                pltpu.sync_copy(ones_v, mask_hbm.at[l, :, gb])

            @pl.when(z != 0)
            def _():
                pltpu.sync_copy(maskt_v.at[:, :, pl.ds(0, 128)],
                                mask_hbm.at[l, :, gb])

        load_slab(0)
        fire(0, 0)

        def pair(t, carry):
            s0 = 2 * t
            s1 = s0 + 1

            @pl.when(lax.rem(s1, _L) != 0)
            def _():
                fire(s1, 1)

            land(s0, 0)
            emit(s0, 0)

            @pl.when(lax.rem(s1, _L) == 0)
            def _():
                load_slab(s1 // _L)
                fire(s1, 1)

            @pl.when((s1 + 1 < _NSTEP) & (lax.rem(s1 + 1, _L) != 0))
            def _():
                fire(s1 + 1, 0)

            land(s1, 1)
            emit(s1, 1)

            @pl.when((s1 + 1 < _NSTEP) & (lax.rem(s1 + 1, _L) == 0))
            def _():
                load_slab((s1 + 1) // _L)
                fire(s1 + 1, 0)

            return carry

        lax.fori_loop(0, _NSTEP // 2, pair, jnp.int32(0))

    return body


_sc_kernel = _make_sc_kernel()


def kernel(x, table):
    xt = jnp.swapaxes(x, 0, 1)
    out5, mask5 = _sc_kernel(xt, table)
    out = out5.transpose(2, 4, 0, 1, 3).reshape(_B, _L, _F)
    mask = mask5.transpose(2, 4, 0, 1, 3).reshape(_B, _L, _F)
    return (out, mask)
